# baseline (speedup 1.0000x reference)
import jax
import jax.numpy as jnp
from jax import lax
from jax.experimental import pallas as pl
from jax.experimental.pallas import tpu as pltpu
from jax.experimental.pallas import tpu_sc as plsc

_NC = 2
_NS = 16
_NW = _NC * _NS
_BB = 128


def _body(idx_hbm, tab_hbm, out_hbm, idx_v, rows_v,
          g0, g1, g2, g3, s0, s1, s2, s3):
    nf = out_hbm.shape[1] // 128
    wid = lax.axis_index("s") * _NC + lax.axis_index("c")
    nunit = idx_v.shape[0] - 2
    u0 = wid * nunit
    pltpu.sync_copy(idx_hbm.at[wid], idx_v)
    gs = [g0, g1, g2, g3]
    ss = [s0, s1, s2, s3]
    bufs = [rows_v.at[i] for i in range(4)]

    def gather(u, b):
        pltpu.async_copy(tab_hbm.at[idx_v.at[u]], bufs[b], gs[b])

    def out_slab(u):
        ug = u0 + u
        bb = ug // nf
        f = ug - bb * nf
        return out_hbm.at[pl.ds(bb * _BB, _BB), pl.ds(f * 128, 128)]

    def wait_gather(b):
        pltpu.make_async_copy(tab_hbm.at[pl.ds(0, _BB)], bufs[b], gs[b]).wait()

    def scatter(u, b):
        pltpu.async_copy(bufs[b], out_slab(u), ss[b])

    def wait_scatter(u, b):
        pltpu.make_async_copy(bufs[b], out_slab(u), ss[b]).wait()

    gather(0, 0)
    gather(1, 1)
    wait_gather(0)
    scatter(0, 0)
    gather(2, 2)
    wait_gather(1)
    scatter(1, 1)
    gather(3, 3)

    @pl.loop(0, (nunit - 4) // 4)
    def _quad(p):
        for k in range(4):
            u = 4 * p + 2 + k
            b = (2 + k) % 4
            wait_gather(b)
            scatter(u, b)
            wait_scatter(u - 2, k % 4)
            gather(u + 2, (2 + k + 2) % 4)

    for k in range(2):
        u = nunit - 2 + k
        b = u % 4
        wait_gather(b)
        scatter(u, b)
        wait_scatter(u - 2, (u - 2) % 4)
        gather(nunit + k, (u + 2) % 4)
    wait_gather(nunit % 4)
    wait_gather((nunit + 1) % 4)
    wait_scatter(nunit - 2, (nunit - 2) % 4)
    wait_scatter(nunit - 1, (nunit - 1) % 4)


def kernel(x, tables):
    b, f = x.shape
    nf, vocab, d = tables.shape
    nbb = b // _BB
    nunits = nbb * nf
    units_per_w = nunits // _NW
    flat_idx = (x.astype(jnp.int32)
                + jnp.arange(nf, dtype=jnp.int32)[None, :] * vocab)
    flat_idx = flat_idx.reshape(nbb, _BB, nf).transpose(0, 2, 1)
    flat_idx = flat_idx.reshape(_NW, units_per_w, _BB)
    flat_idx = jnp.pad(flat_idx, ((0, 0), (0, 2), (0, 0)))
    tab = tables.reshape(nf * vocab, d)
    out = pl.kernel(
        _body,
        out_type=jax.ShapeDtypeStruct((b, f * d), jnp.float32),
        mesh=plsc.VectorSubcoreMesh(core_axis_name="c", subcore_axis_name="s"),
        compiler_params=pltpu.CompilerParams(use_tc_tiling_on_sc=True),
        scratch_types=[
            pltpu.VMEM((units_per_w + 2, _BB), jnp.int32),
            pltpu.VMEM((4, _BB, d), jnp.float32),
        ] + [pltpu.SemaphoreType.DMA] * 8,
    )(flat_idx, tab)
    return out


# speedup vs baseline: 2.7246x; 2.7246x over previous
import jax
import jax.numpy as jnp
from jax import lax
from jax.experimental import pallas as pl
from jax.experimental.pallas import tpu as pltpu
from jax.experimental.pallas import tpu_sc as plsc

_NC = 2
_NS = 16
_NW = _NC * _NS
_BB = 128
_FP = 2


def _body(idx_hbm, tab_hbm, out_hbm, idx_v, rows_v, gsem0, gsem1, ssem0, ssem1):
    nf = out_hbm.shape[1] // 128
    wid = lax.axis_index("s") * _NC + lax.axis_index("c")
    nunit = idx_v.shape[0] // _FP
    u0 = wid * nunit
    pltpu.sync_copy(idx_hbm.at[wid], idx_v)

    def gathers(u, buf, sem):
        for k in range(_FP):
            pltpu.async_copy(tab_hbm.at[idx_v.at[u * _FP + k]],
                             buf.at[:, pl.ds(k * 128, 128)], sem)

    def out_slab(u):
        ug = u0 + u
        nfu = nf // _FP
        bb = ug // nfu
        f0 = (ug - bb * nfu) * _FP
        return out_hbm.at[pl.ds(bb * _BB, _BB), pl.ds(f0 * 128, _FP * 128)]

    def wait_gathers(buf, sem):
        pltpu.make_async_copy(
            out_hbm.at[pl.ds(0, _BB), pl.ds(0, _FP * 128)], buf, sem).wait()

    def scatter(u, buf, sem):
        pltpu.async_copy(buf, out_slab(u), sem)

    def wait_scatter(buf, u, sem):
        pltpu.make_async_copy(buf, out_slab(u), sem).wait()

    buf0, buf1 = rows_v.at[0], rows_v.at[1]

    gathers(0, buf0, gsem0)
    wait_gathers(buf0, gsem0)
    scatter(0, buf0, ssem0)
    gathers(1, buf1, gsem1)

    @pl.loop(0, (nunit - 2) // 2)
    def _pair(p):
        u = 2 * p + 1
        wait_gathers(buf1, gsem1)
        scatter(u, buf1, ssem1)
        wait_scatter(buf0, u - 1, ssem0)
        gathers(u + 1, buf0, gsem0)
        wait_gathers(buf0, gsem0)
        scatter(u + 1, buf0, ssem0)
        wait_scatter(buf1, u, ssem1)
        gathers(u + 2, buf1, gsem1)

    u_last = nunit - 1
    wait_gathers(buf1, gsem1)
    scatter(u_last, buf1, ssem1)
    wait_scatter(buf0, u_last - 1, ssem0)
    wait_scatter(buf1, u_last, ssem1)


def kernel(x, tables):
    b, f = x.shape
    nf, vocab, d = tables.shape
    nbb = b // _BB
    rows_per_w = (nbb * nf * _BB) // _NW
    lists_per_w = rows_per_w // _BB
    flat_idx = (x.astype(jnp.int32)
                + jnp.arange(nf, dtype=jnp.int32)[None, :] * vocab)
    flat_idx = flat_idx.reshape(nbb, _BB, nf).transpose(0, 2, 1)
    flat_idx = flat_idx.reshape(_NW, lists_per_w, _BB)
    tab = tables.reshape(nf * vocab, d)
    out = pl.kernel(
        _body,
        out_type=jax.ShapeDtypeStruct((b, f * d), jnp.float32),
        mesh=plsc.VectorSubcoreMesh(core_axis_name="c", subcore_axis_name="s"),
        compiler_params=pltpu.CompilerParams(use_tc_tiling_on_sc=True),
        scratch_types=[
            pltpu.VMEM((lists_per_w, _BB), jnp.int32),
            pltpu.VMEM((2, _BB, _FP * d), jnp.float32),
            pltpu.SemaphoreType.DMA,
            pltpu.SemaphoreType.DMA,
            pltpu.SemaphoreType.DMA,
            pltpu.SemaphoreType.DMA,
        ],
    )(flat_idx, tab)
    return out


# speedup vs baseline: 2.7320x; 1.0027x over previous
import jax
import jax.numpy as jnp
from jax import lax
from jax.experimental import pallas as pl
from jax.experimental.pallas import tpu as pltpu
from jax.experimental.pallas import tpu_sc as plsc

_NC = 2
_NS = 16
_NW = _NC * _NS
_BB = 128
_FP = 2


def _body(idx_hbm, tab_hbm, out_hbm, idx_v, rows_v,
          gsem0, gsem1, gsem2, ssem0, ssem1, ssem2):
    nf = out_hbm.shape[1] // 128
    wid = lax.axis_index("s") * _NC + lax.axis_index("c")
    nunit = idx_v.shape[0] // _FP
    u0 = wid * nunit
    pltpu.sync_copy(idx_hbm.at[wid], idx_v)

    def gathers(u, buf, sem):
        for k in range(_FP):
            pltpu.async_copy(tab_hbm.at[idx_v.at[u * _FP + k]],
                             buf.at[:, pl.ds(k * 128, 128)], sem)

    def out_slab(u):
        ug = u0 + u
        nfu = nf // _FP
        bb = ug // nfu
        f0 = (ug - bb * nfu) * _FP
        return out_hbm.at[pl.ds(bb * _BB, _BB), pl.ds(f0 * 128, _FP * 128)]

    def wait_gathers(buf, sem):
        pltpu.make_async_copy(
            out_hbm.at[pl.ds(0, _BB), pl.ds(0, _FP * 128)], buf, sem).wait()

    def scatter(u, buf, sem):
        pltpu.async_copy(buf, out_slab(u), sem)

    def wait_scatter(buf, u, sem):
        pltpu.make_async_copy(buf, out_slab(u), sem).wait()

    bufs = [rows_v.at[0], rows_v.at[1], rows_v.at[2]]
    gs = [gsem0, gsem1, gsem2]
    ss = [ssem0, ssem1, ssem2]

    gathers(0, bufs[0], gs[0])
    wait_gathers(bufs[0], gs[0])
    scatter(0, bufs[0], ss[0])
    gathers(1, bufs[1], gs[1])
    wait_gathers(bufs[1], gs[1])
    scatter(1, bufs[1], ss[1])
    gathers(2, bufs[2], gs[2])

    @pl.loop(0, (nunit - 4) // 3)
    def _tri(p):
        for k in range(3):
            u = 3 * p + 2 + k
            b = (2 + k) % 3
            bn = (b + 1) % 3
            wait_gathers(bufs[b], gs[b])
            scatter(u, bufs[b], ss[b])
            wait_scatter(bufs[bn], u - 2, ss[bn])
            gathers(u + 1, bufs[bn], gs[bn])

    for u in (nunit - 2, nunit - 1):
        b = u % 3
        bn = (b + 1) % 3
        wait_gathers(bufs[b], gs[b])
        scatter(u, bufs[b], ss[b])
        if u + 1 < nunit:
            wait_scatter(bufs[bn], u - 2, ss[bn])
            gathers(u + 1, bufs[bn], gs[bn])
    for u in (nunit - 3, nunit - 2, nunit - 1):
        wait_scatter(bufs[u % 3], u, ss[u % 3])


def kernel(x, tables):
    b, f = x.shape
    nf, vocab, d = tables.shape
    nbb = b // _BB
    rows_per_w = (nbb * nf * _BB) // _NW
    lists_per_w = rows_per_w // _BB
    flat_idx = (x.astype(jnp.int32)
                + jnp.arange(nf, dtype=jnp.int32)[None, :] * vocab)
    flat_idx = flat_idx.reshape(nbb, _BB, nf).transpose(0, 2, 1)
    flat_idx = flat_idx.reshape(_NW, lists_per_w, _BB)
    tab = tables.reshape(nf * vocab, d)
    out = pl.kernel(
        _body,
        out_type=jax.ShapeDtypeStruct((b, f * d), jnp.float32),
        mesh=plsc.VectorSubcoreMesh(core_axis_name="c", subcore_axis_name="s"),
        compiler_params=pltpu.CompilerParams(use_tc_tiling_on_sc=True),
        scratch_types=[
            pltpu.VMEM((lists_per_w, _BB), jnp.int32),
            pltpu.VMEM((3, _BB, _FP * d), jnp.float32),
        ] + [pltpu.SemaphoreType.DMA] * 6,
    )(flat_idx, tab)
    return out
